# baseline (speedup 1.0000x reference)
import functools

import jax
import jax.numpy as jnp
from jax.experimental import pallas as pl
from jax.experimental.pallas import tpu as pltpu

_SIZES = [64, 192] + [256] * 62 + [128, 64, 64]
_OFFS = [sum(_SIZES[:i]) for i in range(len(_SIZES))]
_NBLK = len(_SIZES)


def _mm_body(adj_hbm, w_hbm, out_hbm, wv, buf, ob, sem, wsem, osem):
    def cpa(off, rows, slot):
        return pltpu.make_async_copy(
            adj_hbm.at[pl.ds(off, rows), :],
            buf.at[slot, pl.ds(0, rows)],
            sem.at[slot],
        )

    def cpo(off, rows, p):
        return pltpu.make_async_copy(
            ob.at[p, pl.ds(0, rows)],
            out_hbm.at[pl.ds(off, rows), :],
            osem.at[p],
        )

    cpa(_OFFS[0], 64, 0).start()
    pltpu.make_async_copy(w_hbm, wv, wsem).start()
    cpa(_OFFS[1], 192, 1).start()
    cpa(_OFFS[2], 256, 2).start()
    pltpu.make_async_copy(w_hbm, wv, wsem).wait()

    def compute(idx, off, rows, prev_off, prev_rows):
        slot = idx % 3
        p = idx % 2
        cpa(off, rows, slot).wait()
        if idx >= 2:
            cpo(prev_off, prev_rows, p).wait()
        ob[p, pl.ds(0, rows)] = jnp.dot(
            buf[slot, pl.ds(0, rows)], wv[...], preferred_element_type=jnp.float32
        )
        cpo(off, rows, p).start()

    for idx in range(4):
        compute(idx, _OFFS[idx], _SIZES[idx], _OFFS[idx - 2], _SIZES[idx - 2])
        cpa(_OFFS[idx + 3], _SIZES[idx + 3], idx % 3).start()

    def step(i, carry):
        idx = i
        off = (idx - 1) * 256
        slot = jax.lax.rem(idx, 3)
        p = jax.lax.rem(idx, 2)
        pltpu.make_async_copy(
            adj_hbm.at[pl.ds(off, 256), :], buf.at[slot], sem.at[slot]
        ).wait()
        pltpu.make_async_copy(
            ob.at[p], out_hbm.at[pl.ds(off - 512, 256), :], osem.at[p]
        ).wait()
        ob[p] = jnp.dot(buf[slot], wv[...], preferred_element_type=jnp.float32)
        pltpu.make_async_copy(
            ob.at[p], out_hbm.at[pl.ds(off, 256), :], osem.at[p]
        ).start()
        pltpu.make_async_copy(
            adj_hbm.at[pl.ds(off + 768, 256), :], buf.at[slot], sem.at[slot]
        ).start()
        return carry

    jax.lax.fori_loop(4, 61, step, 0)

    for idx in range(61, _NBLK):
        compute(idx, _OFFS[idx], _SIZES[idx], _OFFS[idx - 2], _SIZES[idx - 2])
        nxt = idx + 3
        if nxt < _NBLK:
            cpa(_OFFS[nxt], _SIZES[nxt], nxt % 3).start()

    cpo(_OFFS[_NBLK - 2], _SIZES[_NBLK - 2], (_NBLK - 2) % 2).wait()
    cpo(_OFFS[_NBLK - 1], _SIZES[_NBLK - 1], (_NBLK - 1) % 2).wait()


def kernel(adj, weight):
    m, k = adj.shape
    k2, n = weight.shape
    assert k == k2
    return pl.pallas_call(
        _mm_body,
        in_specs=[
            pl.BlockSpec(memory_space=pltpu.HBM),
            pl.BlockSpec(memory_space=pltpu.HBM),
        ],
        out_specs=pl.BlockSpec(memory_space=pltpu.HBM),
        out_shape=jax.ShapeDtypeStruct((m, n), jnp.float32),
        scratch_shapes=[
            pltpu.VMEM((k2, n), jnp.float32),
            pltpu.VMEM((3, 256, k), jnp.float32),
            pltpu.VMEM((2, 256, n), jnp.float32),
            pltpu.SemaphoreType.DMA((3,)),
            pltpu.SemaphoreType.DMA,
            pltpu.SemaphoreType.DMA((2,)),
        ],
    )(adj, weight)
